# baseline (speedup 1.0000x reference)
import functools

import jax
import jax.numpy as jnp
from jax import lax
from jax.experimental import pallas as pl
from jax.experimental.pallas import tpu as pltpu
from jax.experimental.pallas import tpu_sc as plsc

_N_DATA = 1_000_000
_RHO = 0.99
_ALPHA = 0.5
_EPS = 1e-6
_B = 4096
_NC = 2
_NS = 16
_NW = _NC * _NS
_RANGE = _N_DATA // _NW
_NVEC = _B // 16
_PAD = _B + 16


def _dense_body(feats_ref, c_ref, r_ref, att_ref):
    fa = feats_ref[0:_B, :]
    fb = feats_ref[_B:2 * _B, :]
    fa_roll = jnp.concatenate([fa[1:], fa[:1]], axis=0)
    fb_roll = jnp.concatenate([fb[1:], fb[:1]], axis=0)

    def d2(x):
        return jnp.sum((x * x).reshape(32, 128, 128), axis=2)

    da2 = d2(fa - fb + _EPS)
    db2 = d2(fb - fa + _EPS)
    dra2 = d2(fa - fb_roll + _EPS)
    drb2 = d2(fb - fa_roll + _EPS)
    qa = 1.0 / (1.0 + da2)
    qb = 1.0 / (1.0 + db2)
    qra = 1.0 / (1.0 + dra2)
    qrb = 1.0 / (1.0 + drb2)
    att = (jnp.sum(-jnp.log(qa)) + jnp.sum(-jnp.log(qb))) / (2.0 * _B)
    att_ref[...] = jnp.broadcast_to(att, (1, 1))
    npow2 = jnp.float32(_N_DATA) ** 2
    ema = (1.0 - _RHO) * npow2
    xi_a = _ALPHA * qa + (1.0 - _ALPHA) * qra
    xi_b = _ALPHA * qb + (1.0 - _ALPHA) * qrb
    c_ref[...] = (ema * xi_a + ema * xi_b) * 0.5
    r_ref[...] = qra + qrb


_dense_call = pl.pallas_call(
    _dense_body,
    out_shape=(
        jax.ShapeDtypeStruct((32, 128), jnp.float32),
        jax.ShapeDtypeStruct((32, 128), jnp.float32),
        jax.ShapeDtypeStruct((1, 1), jnp.float32),
    ),
)


_sc_mesh = plsc.VectorSubcoreMesh(
    core_axis_name="c", subcore_axis_name="s", num_cores=_NC, num_subcores=_NS
)


@functools.partial(
    pl.kernel,
    out_type=jax.ShapeDtypeStruct((_NW * 16,), jnp.float32),
    mesh=_sc_mesh,
    compiler_params=pltpu.CompilerParams(needs_layout_passes=False),
    scratch_types=[
        pltpu.VMEM((_B,), jnp.int32),
        pltpu.VMEM((_B,), jnp.float32),
        pltpu.VMEM((_B,), jnp.float32),
        pltpu.VMEM((_PAD,), jnp.int32),
        pltpu.VMEM((_PAD,), jnp.int32),
        pltpu.VMEM((_PAD,), jnp.float32),
        pltpu.VMEM((_PAD,), jnp.float32),
        pltpu.VMEM((_RANGE,), jnp.int32),
        pltpu.VMEM((16,), jnp.float32),
        pltpu.SemaphoreType.DMA,
        pltpu.SemaphoreType.DMA,
        pltpu.SemaphoreType.DMA,
    ],
)
def _sc_update(s_inv_hbm, idx_hbm, c_hbm, r_hbm, sref_hbm, parts_hbm,
               idx_v, c_v, r_v, cidx_v, cpos_v, scur_v, val_v, stamp_v,
               part_v, isem, gsem, ssem):
    wid = lax.axis_index("s") * _NC + lax.axis_index("c")
    obase = wid * _RANGE
    iota = lax.iota(jnp.int32, 16)

    d1 = pltpu.async_copy(idx_hbm, idx_v, isem)
    d2 = pltpu.async_copy(c_hbm, c_v, isem)
    d3 = pltpu.async_copy(r_hbm, r_v, isem)
    d1.wait()
    d2.wait()
    d3.wait()

    def scan_body(i, cnt):
        sl = pl.ds(i * 16, 16)
        off = idx_v[sl] - obase
        inr = (off >= 0) & (off < _RANGE)
        plsc.store_compressed(cidx_v.at[pl.ds(cnt, 16)], idx_v[sl], mask=inr)
        plsc.store_compressed(cpos_v.at[pl.ds(cnt, 16)], iota + i * 16, mask=inr)
        return cnt + jnp.sum(inr.astype(jnp.int32))

    cnt = lax.fori_loop(0, _NVEC, scan_body, jnp.int32(0))
    cidx_v[pl.ds(cnt, 16)] = jnp.broadcast_to(obase, (16,))
    cpos_v[pl.ds(cnt, 16)] = jnp.zeros((16,), jnp.int32)
    nch = (cnt + 15) // 16

    def fire_g(j, _):
        ivec = cidx_v[pl.ds(j * 16, 16)]
        pltpu.async_copy(s_inv_hbm.at[ivec], scur_v.at[pl.ds(j * 16, 16)], gsem)
        return 0

    lax.fori_loop(0, nch, fire_g, 0)

    def drain_g(j, _):
        pltpu.make_async_copy(
            s_inv_hbm.at[pl.ds(0, 16)], scur_v.at[pl.ds(j * 16, 16)], gsem
        ).wait()
        return 0

    lax.fori_loop(0, nch, drain_g, 0)

    def comp_body(j, acc):
        sl = pl.ds(j * 16, 16)
        s_cur = scur_v[sl]
        posv = cpos_v[sl]
        cval = plsc.load_gather(c_v, [posv])
        rval = plsc.load_gather(r_v, [posv])
        valid = (iota + j * 16) < cnt
        val_v[sl] = _RHO * s_cur + cval
        return acc + jnp.where(valid, rval / s_cur, 0.0)

    acc = lax.fori_loop(0, nch, comp_body, jnp.zeros((16,), jnp.float32))

    def stamp_body(j, _):
        sl = pl.ds(j * 16, 16)
        ivc = cidx_v[sl]
        valid = (iota + j * 16) < cnt
        _, last = plsc.scan_count(ivc, mask=valid)
        m1 = valid & last
        plsc.store_scatter(stamp_v, [ivc - obase], iota + j * 16, mask=m1)
        return 0

    lax.fori_loop(0, nch, stamp_body, 0)

    def scat_body(j, nf):
        sl = pl.ds(j * 16, 16)
        ivc = cidx_v[sl]
        posl = iota + j * 16
        valid = posl < cnt
        _, last = plsc.scan_count(ivc, mask=valid)
        m1 = valid & last
        stamped = plsc.load_gather(stamp_v, [ivc - obase], mask=m1)
        win = m1 & (stamped == posl)
        anyw = jnp.any(win)

        @pl.when(anyw)
        def _():
            vv = val_v[sl]
            wpos = jnp.max(jnp.where(win, iota, -1))
            sel = iota == wpos
            bidx = jnp.sum(jnp.where(sel, ivc, 0))
            bval = jnp.sum(jnp.where(sel, vv, 0.0))
            sidx = jnp.where(win, ivc, bidx)
            val_v[sl] = jnp.where(win, vv, bval)
            pltpu.async_copy(val_v.at[sl], sref_hbm.at[sidx], ssem)

        return nf + anyw.astype(jnp.int32)

    nf = lax.fori_loop(0, nch, scat_body, jnp.int32(0))

    def drain_s(j, _):
        pltpu.make_async_copy(
            s_inv_hbm.at[pl.ds(0, 16)], val_v.at[pl.ds(0, 16)], ssem
        ).wait()
        return 0

    lax.fori_loop(0, nf, drain_s, 0)

    part_v[...] = jnp.broadcast_to(jnp.sum(acc), (16,))
    pltpu.sync_copy(part_v, parts_hbm.at[pl.ds(wid * 16, 16)])


def kernel(feats, feats_idx, s_inv):
    c2, r2, att = _dense_call(feats)
    sref = jax.new_ref(s_inv)
    parts = _sc_update(s_inv, feats_idx, c2.reshape(_B), r2.reshape(_B), sref)
    new_s_inv = sref[...]
    npow2 = jnp.float32(_N_DATA) ** 2
    rep = jnp.sum(parts.reshape(_NW, 16)[:, 0]) * (npow2 / jnp.float32(2 * _B))
    loss = att[0, 0] + rep
    return loss, new_s_inv
